# baseline (speedup 1.0000x reference)
import jax
import jax.numpy as jnp
from jax.experimental import pallas as pl
from jax.experimental.pallas import tpu as pltpu

N = 10000
D = 64
BM = 80


def _matmul_block(a_ref, e_ref, o_ref):
    o_ref[...] = jnp.dot(
        a_ref[...], e_ref[...], preferred_element_type=jnp.float32
    )


def kernel(A_hat, E):
    return pl.pallas_call(
        _matmul_block,
        grid=(N // BM,),
        in_specs=[
            pl.BlockSpec((BM, N), lambda i: (i, 0)),
            pl.BlockSpec((N, D), lambda i: (0, 0)),
        ],
        out_specs=pl.BlockSpec((BM, D), lambda i: (i, 0)),
        out_shape=jax.ShapeDtypeStruct((N, D), jnp.float32),
        compiler_params=pltpu.CompilerParams(
            dimension_semantics=("parallel",),
        ),
    )(A_hat, E)


# speedup vs baseline: 1.3324x; 1.3324x over previous
import jax
import jax.numpy as jnp
from jax.experimental import pallas as pl
from jax.experimental.pallas import tpu as pltpu

N = 10000
D = 64
BM = 200


def _matmul_block(a_ref, e_ref, o_ref):
    o_ref[...] = jnp.dot(
        a_ref[...], e_ref[...], preferred_element_type=jnp.float32
    )


def kernel(A_hat, E):
    return pl.pallas_call(
        _matmul_block,
        grid=(N // BM,),
        in_specs=[
            pl.BlockSpec((BM, N), lambda i: (i, 0)),
            pl.BlockSpec((N, D), lambda i: (0, 0)),
        ],
        out_specs=pl.BlockSpec((BM, D), lambda i: (i, 0)),
        out_shape=jax.ShapeDtypeStruct((N, D), jnp.float32),
    )(A_hat, E)


# speedup vs baseline: 1.3359x; 1.0026x over previous
import jax
import jax.numpy as jnp
from jax.experimental import pallas as pl
from jax.experimental.pallas import tpu as pltpu

N = 10000
D = 64
BM = 200


def _matmul_block(a_ref, e_ref, o_ref):
    o_ref[...] = jnp.dot(
        a_ref[...], e_ref[...], preferred_element_type=jnp.float32
    )


def kernel(A_hat, E):
    return pl.pallas_call(
        _matmul_block,
        grid=(N // BM,),
        in_specs=[
            pl.BlockSpec((BM, N), lambda i: (i, 0)),
            pl.BlockSpec((N, D), lambda i: (0, 0)),
        ],
        out_specs=pl.BlockSpec((BM, D), lambda i: (i, 0)),
        out_shape=jax.ShapeDtypeStruct((N, D), jnp.float32),
        compiler_params=pltpu.CompilerParams(
            dimension_semantics=("parallel",),
        ),
    )(A_hat, E)
